# Initial kernel scaffold; baseline (speedup 1.0000x reference)
import jax
import jax.numpy as jnp
from jax.experimental import pallas as pl


def kernel(x, edge_index, edge_type, emb, lins_w, films_w, films_b, skip_w, film_skip_w, bn_w, bn_b, lin1_w, lin1_b, lin2_w, lin2_b):
    raise NotImplementedError("write your pallas kernel here")


# speedup vs baseline: 7.7490x; 7.7490x over previous
import functools

import jax
import jax.numpy as jnp
from jax import lax
from jax.experimental import pallas as pl
from jax.experimental.pallas import tpu as pltpu
from jax.experimental.pallas import tpu_sc as plsc

N_NODES = 10000
N_EDGES = 320000
HID = 128
N_REL = 4
EPS = 1e-5

NC = 2
NS = 16
NW = NC * NS
EPW = N_EDGES // NW
K = 80
NCHUNK = EPW // K
CNT_PAD = 40960
CSLICE = CNT_PAD // NS
RPT = N_NODES // NS
NPW = 312
E_CNT = N_EDGES // NS

_mesh = plsc.VectorSubcoreMesh(core_axis_name="c", subcore_axis_name="s")


def _f32(shape):
    return jax.ShapeDtypeStruct(shape, jnp.float32)


def _i32(shape):
    return jax.ShapeDtypeStruct(shape, jnp.int32)


@functools.partial(
    pl.kernel,
    out_type=(
        _f32((N_NODES, HID)),
        _i32((N_EDGES,)),
        _i32((N_EDGES,)),
        _f32((CNT_PAD,)),
    ),
    mesh=_mesh,
    scratch_types=[
        pltpu.VMEM((NPW + 16,), jnp.int32),
        pltpu.VMEM((NPW + 16, HID), jnp.float32),
        pltpu.VMEM((K,), jnp.int32),
        pltpu.VMEM((K,), jnp.int32),
        pltpu.VMEM((K,), jnp.int32),
        pltpu.VMEM((K,), jnp.int32),
        pltpu.VMEM((K,), jnp.int32),
        pltpu.VMEM((K,), jnp.float32),
        pltpu.VMEM((CSLICE,), jnp.float32),
        pltpu.VMEM_SHARED((CNT_PAD,), jnp.float32),
        pltpu.SemaphoreType.DMA,
    ],
)
def _sc_prep(src_hbm, dst_hbm, et_hbm, x_hbm, emb_hbm,
             h0_hbm, hridx_hbm, gbidx_hbm, inv_hbm,
             xv, rows, srcv, dstv, typev, hrbuf, gbbuf, onesv, cslice,
             cnt_sh, sem):
    c = lax.axis_index("c")
    s = lax.axis_index("s")
    wid = c * NS + s

    def zb(i, _):
        cslice[pl.ds(i * 16, 16)] = jnp.zeros((16,), jnp.float32)
        return 0
    lax.fori_loop(0, CSLICE // 16, zb, 0)
    pltpu.sync_copy(cslice, cnt_sh.at[pl.ds(s * CSLICE, CSLICE)])

    def ob(i, _):
        onesv[pl.ds(i * 16, 16)] = jnp.ones((16,), jnp.float32)
        return 0
    lax.fori_loop(0, K // 16, ob, 0)

    nbase = wid * NPW
    pltpu.sync_copy(x_hbm.at[pl.ds(nbase, NPW)], xv.at[pl.ds(0, NPW)])
    pltpu.async_copy(emb_hbm.at[xv.at[pl.ds(0, NPW)]],
                     rows.at[pl.ds(0, NPW)], sem).wait()
    pltpu.sync_copy(rows.at[pl.ds(0, NPW)], h0_hbm.at[pl.ds(nbase, NPW)])

    @pl.when(wid == 0)
    def _():
        tail = N_NODES - NW * NPW
        tb = NW * NPW
        pltpu.sync_copy(x_hbm.at[pl.ds(tb, tail)], xv.at[pl.ds(0, tail)])
        pltpu.async_copy(emb_hbm.at[xv.at[pl.ds(0, tail)]],
                         rows.at[pl.ds(0, tail)], sem).wait()
        pltpu.sync_copy(rows.at[pl.ds(0, tail)], h0_hbm.at[pl.ds(tb, tail)])

    ebase = wid * EPW

    def chunk(i, _):
        base = ebase + i * K
        pltpu.sync_copy(src_hbm.at[pl.ds(base, K)], srcv)
        pltpu.sync_copy(dst_hbm.at[pl.ds(base, K)], dstv)
        pltpu.sync_copy(et_hbm.at[pl.ds(base, K)], typev)

        def sub(j, _):
            sl = pl.ds(j * 16, 16)
            t = typev[sl]
            hrbuf[sl] = t * N_NODES + srcv[sl]
            gbbuf[sl] = t * N_NODES + dstv[sl]
            return 0
        lax.fori_loop(0, K // 16, sub, 0)
        pltpu.sync_copy(hrbuf, hridx_hbm.at[pl.ds(base, K)])
        pltpu.sync_copy(gbbuf, gbidx_hbm.at[pl.ds(base, K)])
        return 0
    lax.fori_loop(0, NCHUNK, chunk, 0)

    plsc.subcore_barrier()

    @pl.when(c == 0)
    def _():
        cbase = s * E_CNT

        def cchunk(i, _):
            base = cbase + i * K
            pltpu.sync_copy(dst_hbm.at[pl.ds(base, K)], dstv)
            pltpu.sync_copy(et_hbm.at[pl.ds(base, K)], typev)

            def sub(j, _):
                sl = pl.ds(j * 16, 16)
                gbbuf[sl] = typev[sl] * N_NODES + dstv[sl]
                return 0
            lax.fori_loop(0, K // 16, sub, 0)
            pltpu.sync_copy(onesv, cnt_sh.at[gbbuf], add=True)
            return 0
        lax.fori_loop(0, E_CNT // K, cchunk, 0)

    plsc.subcore_barrier()

    @pl.when(c == 0)
    def _():
        pltpu.sync_copy(cnt_sh.at[pl.ds(s * CSLICE, CSLICE)], cslice)

        def ib(i, _):
            sl = pl.ds(i * 16, 16)
            cslice[sl] = 1.0 / jnp.maximum(cslice[sl], 1.0)
            return 0
        lax.fori_loop(0, CSLICE // 16, ib, 0)
        pltpu.sync_copy(cslice, inv_hbm.at[pl.ds(s * CSLICE, CSLICE)])


@functools.partial(
    pl.kernel,
    out_type=_f32((NC, N_NODES, HID)),
    mesh=_mesh,
    scratch_types=[
        pltpu.VMEM((K, HID), jnp.float32),
        pltpu.VMEM((K, 2 * HID), jnp.float32),
        pltpu.VMEM((K, HID), jnp.float32),
        pltpu.VMEM((K,), jnp.int32),
        pltpu.VMEM((K,), jnp.int32),
        pltpu.VMEM((K,), jnp.int32),
        pltpu.VMEM((K,), jnp.float32),
        pltpu.VMEM((16, HID), jnp.float32),
        pltpu.VMEM_SHARED((N_NODES, HID), jnp.float32),
        pltpu.SemaphoreType.DMA,
        pltpu.SemaphoreType.DMA,
        pltpu.SemaphoreType.DMA,
    ],
)
def _sc_edge(hr_hbm, gb_hbm, hridx_hbm, gbidx_hbm, dst_hbm, inv_hbm,
             out_hbm,
             hrv, gbv, msgv, hridxv, gbidxv, dstv, invv, zrow,
             acc_sh, sem, sem2, sem3):
    c = lax.axis_index("c")
    s = lax.axis_index("s")
    wid = c * NS + s

    rbase = s * 624
    nch = jnp.where(s == NS - 1, 40, 39)

    def zfill(r, _):
        for j in range(HID // 16):
            zrow[r, pl.ds(j * 16, 16)] = jnp.zeros((16,), jnp.float32)
        return 0
    lax.fori_loop(0, 16, zfill, 0)

    def zcopy(q, _):
        pltpu.sync_copy(zrow, acc_sh.at[pl.ds(rbase + q * 16, 16)])
        return 0
    lax.fori_loop(0, nch, zcopy, 0)
    plsc.subcore_barrier()

    ebase = wid * EPW

    def chunk(i, _):
        base = ebase + i * K
        pltpu.sync_copy(hridx_hbm.at[pl.ds(base, K)], hridxv)
        pltpu.sync_copy(gbidx_hbm.at[pl.ds(base, K)], gbidxv)
        pltpu.sync_copy(dst_hbm.at[pl.ds(base, K)], dstv)
        cp1 = pltpu.async_copy(hr_hbm.at[hridxv], hrv, sem)
        cp2 = pltpu.async_copy(gb_hbm.at[gbidxv], gbv, sem2)
        cp3 = pltpu.async_copy(inv_hbm.at[gbidxv], invv, sem3)
        cp1.wait()
        cp2.wait()
        cp3.wait()

        def ebody(g, _):
            inv16 = invv[pl.ds(g * 16, 16)]
            for ee in range(16):
                e = g * 16 + ee
                iv = inv16[ee]
                for j in range(HID // 16):
                    sl = pl.ds(j * 16, 16)
                    beta = gbv[e, sl]
                    gamma = gbv[e, pl.ds(HID + j * 16, 16)]
                    hr = hrv[e, sl]
                    msgv[e, sl] = jnp.maximum(gamma * hr + beta, 0.0) * iv
            return 0
        lax.fori_loop(0, K // 16, ebody, 0)

        pltpu.sync_copy(msgv, acc_sh.at[dstv], add=True)
        return 0
    lax.fori_loop(0, NCHUNK, chunk, 0)

    plsc.subcore_barrier()

    def ocopy(q, _):
        rb = rbase + q * 16
        pltpu.sync_copy(acc_sh.at[pl.ds(rb, 16)], zrow)
        pltpu.sync_copy(zrow, out_hbm.at[c, pl.ds(rb, 16)])
        return 0
    lax.fori_loop(0, nch, ocopy, 0)


_NB = 1000


def _tc_proj_body(h_ref, lw_ref, fw_ref, fb_ref, sw_ref, fsw_ref,
                  hr_ref, gb_ref, so_ref):
    h = h_ref[...]
    for r in range(N_REL):
        hr_ref[r] = jnp.dot(h, lw_ref[r], preferred_element_type=jnp.float32)
        gb_ref[r] = (jnp.dot(h, fw_ref[r], preferred_element_type=jnp.float32)
                     + fb_ref[r][None, :])
    bgs = jnp.dot(h, fsw_ref[...], preferred_element_type=jnp.float32)
    hs = jnp.dot(h, sw_ref[...], preferred_element_type=jnp.float32)
    so_ref[...] = jnp.maximum(bgs[:, HID:] * hs + bgs[:, :HID], 0.0)


def _tc_proj(h, lw, fw, fb, sw, fsw):
    grid = N_NODES // _NB
    return pl.pallas_call(
        _tc_proj_body,
        grid=(grid,),
        in_specs=[
            pl.BlockSpec((_NB, HID), lambda i: (i, 0)),
            pl.BlockSpec((N_REL, HID, HID), lambda i: (0, 0, 0)),
            pl.BlockSpec((N_REL, HID, 2 * HID), lambda i: (0, 0, 0)),
            pl.BlockSpec((N_REL, 2 * HID), lambda i: (0, 0)),
            pl.BlockSpec((HID, HID), lambda i: (0, 0)),
            pl.BlockSpec((HID, 2 * HID), lambda i: (0, 0)),
        ],
        out_specs=[
            pl.BlockSpec((N_REL, _NB, HID), lambda i: (0, i, 0)),
            pl.BlockSpec((N_REL, _NB, 2 * HID), lambda i: (0, i, 0)),
            pl.BlockSpec((_NB, HID), lambda i: (i, 0)),
        ],
        out_shape=[
            _f32((N_REL, N_NODES, HID)),
            _f32((N_REL, N_NODES, 2 * HID)),
            _f32((N_NODES, HID)),
        ],
    )(h, lw, fw, fb, sw, fsw)


def _bn(hsum, w, b):
    m = jnp.mean(hsum, axis=0, keepdims=True)
    v = jnp.mean((hsum - m) * (hsum - m), axis=0, keepdims=True)
    return (hsum - m) * jax.lax.rsqrt(v + EPS) * w + b


def _tc_combine_body(so_ref, part_ref, w_ref, b_ref, out_ref):
    hsum = so_ref[...] + part_ref[0] + part_ref[1]
    out_ref[...] = _bn(hsum, w_ref[...], b_ref[...])


def _tc_combine(so, parts, w, b):
    return pl.pallas_call(
        _tc_combine_body,
        out_shape=_f32((N_NODES, HID)),
    )(so, parts, w, b)


def _tc_final_body(so_ref, part_ref, w_ref, b_ref, l1w_ref, l1b_ref,
                   l2w_ref, l2b_ref, out_ref):
    hsum = so_ref[...] + part_ref[0] + part_ref[1]
    h2 = _bn(hsum, w_ref[...], b_ref[...])
    t = jnp.dot(h2, l1w_ref[...], preferred_element_type=jnp.float32)
    t = t + l1b_ref[...]
    t = jnp.where(t > 0, t, 0.2 * t)
    out_ref[...] = (jnp.dot(t, l2w_ref[...],
                            preferred_element_type=jnp.float32)
                    + l2b_ref[...])


def _tc_final(so, parts, w, b, l1w, l1b, l2w, l2b):
    return pl.pallas_call(
        _tc_final_body,
        out_shape=_f32((N_NODES, l2w.shape[1])),
    )(so, parts, w, b, l1w, l1b, l2w, l2b)


def kernel(x, edge_index, edge_type, emb, lins_w, films_w, films_b,
           skip_w, film_skip_w, bn_w, bn_b, lin1_w, lin1_b, lin2_w, lin2_b):
    src = edge_index[0]
    dst = edge_index[1]
    h0, hridx, gbidx, inv_cnt = _sc_prep(src, dst, edge_type, x, emb)

    h = h0
    out = None
    n_layers = lins_w.shape[0]
    for l in range(n_layers):
        hr_all, gb_all, so = _tc_proj(h, lins_w[l], films_w[l], films_b[l],
                                      skip_w[l], film_skip_w[l])
        parts = _sc_edge(hr_all.reshape(N_REL * N_NODES, HID),
                         gb_all.reshape(N_REL * N_NODES, 2 * HID),
                         hridx, gbidx, dst, inv_cnt)
        wl = bn_w[l].reshape(1, HID)
        bl = bn_b[l].reshape(1, HID)
        if l < n_layers - 1:
            h = _tc_combine(so, parts, wl, bl)
        else:
            out = _tc_final(so, parts, wl, bl, lin1_w,
                            lin1_b.reshape(1, -1), lin2_w,
                            lin2_b.reshape(1, -1))
    return out


# speedup vs baseline: 9.3830x; 1.2109x over previous
import functools

import jax
import jax.numpy as jnp
from jax import lax
from jax.experimental import pallas as pl
from jax.experimental.pallas import tpu as pltpu
from jax.experimental.pallas import tpu_sc as plsc

N_NODES = 10000
N_EDGES = 320000
HID = 128
N_REL = 4
EPS = 1e-5

NC = 2
NS = 16
NW = NC * NS
EPW = N_EDGES // NW
K = 40
NCHUNK = EPW // K
CNT_PAD = 40960
CSLICE = CNT_PAD // NS
RPT = N_NODES // NS
NPW = 312
E_CNT = N_EDGES // NS

_mesh = plsc.VectorSubcoreMesh(core_axis_name="c", subcore_axis_name="s")


def _spans16(n):
    s = [i * 16 for i in range(n // 16)]
    if n % 16:
        s.append(n - 16)
    return s


def _f32(shape):
    return jax.ShapeDtypeStruct(shape, jnp.float32)


def _i32(shape):
    return jax.ShapeDtypeStruct(shape, jnp.int32)


@functools.partial(
    pl.kernel,
    out_type=(
        _f32((N_NODES, HID)),
        _i32((N_EDGES,)),
        _i32((N_EDGES,)),
        _f32((CNT_PAD,)),
    ),
    mesh=_mesh,
    scratch_types=[
        pltpu.VMEM((NPW + 16,), jnp.int32),
        pltpu.VMEM((NPW + 16, HID), jnp.float32),
        pltpu.VMEM((K,), jnp.int32),
        pltpu.VMEM((K,), jnp.int32),
        pltpu.VMEM((K,), jnp.int32),
        pltpu.VMEM((K,), jnp.int32),
        pltpu.VMEM((K,), jnp.int32),
        pltpu.VMEM((K,), jnp.float32),
        pltpu.VMEM((CSLICE,), jnp.float32),
        pltpu.VMEM_SHARED((CNT_PAD,), jnp.float32),
        pltpu.SemaphoreType.DMA,
    ],
)
def _sc_prep(src_hbm, dst_hbm, et_hbm, x_hbm, emb_hbm,
             h0_hbm, hridx_hbm, gbidx_hbm, inv_hbm,
             xv, rows, srcv, dstv, typev, hrbuf, gbbuf, onesv, cslice,
             cnt_sh, sem):
    c = lax.axis_index("c")
    s = lax.axis_index("s")
    wid = c * NS + s

    def zb(i, _):
        cslice[pl.ds(i * 16, 16)] = jnp.zeros((16,), jnp.float32)
        return 0
    lax.fori_loop(0, CSLICE // 16, zb, 0)
    pltpu.sync_copy(cslice, cnt_sh.at[pl.ds(s * CSLICE, CSLICE)])

    for g0 in _spans16(K):
        onesv[pl.ds(g0, 16)] = jnp.ones((16,), jnp.float32)

    nbase = wid * NPW
    pltpu.sync_copy(x_hbm.at[pl.ds(nbase, NPW)], xv.at[pl.ds(0, NPW)])
    pltpu.async_copy(emb_hbm.at[xv.at[pl.ds(0, NPW)]],
                     rows.at[pl.ds(0, NPW)], sem).wait()
    pltpu.sync_copy(rows.at[pl.ds(0, NPW)], h0_hbm.at[pl.ds(nbase, NPW)])

    @pl.when(wid == 0)
    def _():
        tail = N_NODES - NW * NPW
        tb = NW * NPW
        pltpu.sync_copy(x_hbm.at[pl.ds(tb, tail)], xv.at[pl.ds(0, tail)])
        pltpu.async_copy(emb_hbm.at[xv.at[pl.ds(0, tail)]],
                         rows.at[pl.ds(0, tail)], sem).wait()
        pltpu.sync_copy(rows.at[pl.ds(0, tail)], h0_hbm.at[pl.ds(tb, tail)])

    ebase = wid * EPW

    def chunk(i, _):
        base = ebase + i * K
        pltpu.sync_copy(src_hbm.at[pl.ds(base, K)], srcv)
        pltpu.sync_copy(dst_hbm.at[pl.ds(base, K)], dstv)
        pltpu.sync_copy(et_hbm.at[pl.ds(base, K)], typev)

        for g0 in _spans16(K):
            sl = pl.ds(g0, 16)
            t = typev[sl]
            hrbuf[sl] = t * N_NODES + srcv[sl]
            gbbuf[sl] = t * N_NODES + dstv[sl]
        pltpu.sync_copy(hrbuf, hridx_hbm.at[pl.ds(base, K)])
        pltpu.sync_copy(gbbuf, gbidx_hbm.at[pl.ds(base, K)])
        return 0
    lax.fori_loop(0, NCHUNK, chunk, 0)

    plsc.subcore_barrier()

    @pl.when(c == 0)
    def _():
        cbase = s * E_CNT

        def cchunk(i, _):
            base = cbase + i * K
            pltpu.sync_copy(dst_hbm.at[pl.ds(base, K)], dstv)
            pltpu.sync_copy(et_hbm.at[pl.ds(base, K)], typev)

            for g0 in _spans16(K):
                sl = pl.ds(g0, 16)
                gbbuf[sl] = typev[sl] * N_NODES + dstv[sl]
            pltpu.sync_copy(onesv, cnt_sh.at[gbbuf], add=True)
            return 0
        lax.fori_loop(0, E_CNT // K, cchunk, 0)

    plsc.subcore_barrier()

    @pl.when(c == 0)
    def _():
        pltpu.sync_copy(cnt_sh.at[pl.ds(s * CSLICE, CSLICE)], cslice)

        def ib(i, _):
            sl = pl.ds(i * 16, 16)
            cslice[sl] = 1.0 / jnp.maximum(cslice[sl], 1.0)
            return 0
        lax.fori_loop(0, CSLICE // 16, ib, 0)
        pltpu.sync_copy(cslice, inv_hbm.at[pl.ds(s * CSLICE, CSLICE)])


@functools.partial(
    pl.kernel,
    out_type=_f32((NC, N_NODES, HID)),
    mesh=_mesh,
    scratch_types=[
        pltpu.VMEM((K, HID), jnp.float32),
        pltpu.VMEM((K, HID), jnp.float32),
        pltpu.VMEM((K, 2 * HID), jnp.float32),
        pltpu.VMEM((K, 2 * HID), jnp.float32),
        pltpu.VMEM((K, HID), jnp.float32),
        pltpu.VMEM((K, HID), jnp.float32),
        pltpu.VMEM((K,), jnp.int32),
        pltpu.VMEM((K,), jnp.int32),
        pltpu.VMEM((K,), jnp.int32),
        pltpu.VMEM((K,), jnp.int32),
        pltpu.VMEM((K,), jnp.int32),
        pltpu.VMEM((K,), jnp.int32),
        pltpu.VMEM((K,), jnp.int32),
        pltpu.VMEM((K,), jnp.int32),
        pltpu.VMEM((16, HID), jnp.float32),
        pltpu.VMEM_SHARED((N_NODES, HID), jnp.float32),
        pltpu.SemaphoreType.DMA,
        pltpu.SemaphoreType.DMA,
        pltpu.SemaphoreType.DMA,
        pltpu.SemaphoreType.DMA,
        pltpu.SemaphoreType.DMA,
        pltpu.SemaphoreType.DMA,
        pltpu.SemaphoreType.DMA,
        pltpu.SemaphoreType.DMA,
    ],
)
def _sc_edge(hr_hbm, gb_hbm, hridx_hbm, gbidx_hbm, dst_hbm,
             out_hbm,
             hrv0, hrv1, gbv0, gbv1, msgv0, msgv1,
             hrS0, hrS1, gbS0, gbS1, dstS0, dstS1, dstv0, dstv1,
             zrow, acc_sh,
             shr0, shr1, sgb0, sgb1, sidx0, sidx1, ssc0, ssc1):
    c = lax.axis_index("c")
    s = lax.axis_index("s")
    wid = c * NS + s
    bufs = (
        (hrv0, gbv0, msgv0, hrS0, gbS0, dstS0, dstv0, shr0, sgb0, sidx0,
         ssc0),
        (hrv1, gbv1, msgv1, hrS1, gbS1, dstS1, dstv1, shr1, sgb1, sidx1,
         ssc1),
    )

    rbase = s * 624
    nch = jnp.where(s == NS - 1, 40, 39)

    ebase = wid * EPW

    def zfill(r, _):
        for j in range(HID // 16):
            zrow[r, pl.ds(j * 16, 16)] = jnp.zeros((16,), jnp.float32)
        return 0
    lax.fori_loop(0, 16, zfill, 0)

    def zcopy(q, _):
        pltpu.sync_copy(zrow, acc_sh.at[pl.ds(rbase + q * 16, 16)])
        return 0
    lax.fori_loop(0, nch, zcopy, 0)
    plsc.subcore_barrier()

    def issue_idx(i, b):
        _, _, _, hrS, gbS, dstS, _, _, _, sidx, _ = bufs[b]
        sl = pl.ds(ebase + i * K, K)
        pltpu.async_copy(hridx_hbm.at[sl], hrS, sidx)
        pltpu.async_copy(gbidx_hbm.at[sl], gbS, sidx)
        pltpu.async_copy(dst_hbm.at[sl], dstS, sidx)

    def wait_idx(i, b):
        _, _, _, hrS, gbS, dstS, _, _, _, sidx, _ = bufs[b]
        sl = pl.ds(ebase + i * K, K)
        pltpu.make_async_copy(hridx_hbm.at[sl], hrS, sidx).wait()
        pltpu.make_async_copy(gbidx_hbm.at[sl], gbS, sidx).wait()
        pltpu.make_async_copy(dst_hbm.at[sl], dstS, sidx).wait()

    def issue_gathers(b):
        hrv, gbv, _, hrS, gbS, _, _, shr, sgb, _, _ = bufs[b]
        pltpu.async_copy(hr_hbm.at[hrS], hrv, shr)
        pltpu.async_copy(gb_hbm.at[gbS], gbv, sgb)

    def wait_gathers(b):
        hrv, gbv, _, hrS, gbS, _, _, shr, sgb, _, _ = bufs[b]
        pltpu.make_async_copy(hr_hbm.at[hrS], hrv, shr).wait()
        pltpu.make_async_copy(gb_hbm.at[gbS], gbv, sgb).wait()

    def process(i, b, pre_idx=True, pre_gather=True):
        hrv, gbv, msgv, hrS, gbS, dstS, dstv, shr, sgb, sidx, ssc = bufs[b]
        if pre_gather:
            wait_idx(i + 1, 1 - b)
            issue_gathers(1 - b)
        wait_gathers(b)
        for g0 in _spans16(K):
            gsl = pl.ds(g0, 16)
            dstv[gsl] = dstS[gsl]
        if pre_idx:
            issue_idx(i + 2, b)

        def ebody(e, _):
            for j in range(HID // 16):
                fsl = pl.ds(j * 16, 16)
                beta = gbv[e, fsl]
                gamma = gbv[e, pl.ds(HID + j * 16, 16)]
                hr = hrv[e, fsl]
                msgv[e, fsl] = jnp.maximum(gamma * hr + beta, 0.0)
            return 0
        lax.fori_loop(0, K, ebody, 0)

        pltpu.sync_copy(msgv, acc_sh.at[dstv], add=True)

    issue_idx(0, 0)
    wait_idx(0, 0)
    issue_gathers(0)
    issue_idx(1, 1)

    def pair(ip, _):
        process(2 * ip, 0)
        process(2 * ip + 1, 1)
        return 0
    lax.fori_loop(0, NCHUNK // 2 - 1, pair, 0)

    process(NCHUNK - 2, 0, pre_idx=False, pre_gather=True)
    process(NCHUNK - 1, 1, pre_idx=False, pre_gather=False)

    plsc.subcore_barrier()

    def ocopy(q, _):
        rb = rbase + q * 16
        pltpu.sync_copy(acc_sh.at[pl.ds(rb, 16)], zrow)
        pltpu.sync_copy(zrow, out_hbm.at[c, pl.ds(rb, 16)])
        return 0
    lax.fori_loop(0, nch, ocopy, 0)


_NB = 1000


def _tc_proj_body(h_ref, lw_ref, fw_ref, fb_ref, sw_ref, fsw_ref, inv_ref,
                  hr_ref, gb_ref, so_ref):
    h = h_ref[...]
    for r in range(N_REL):
        hr_ref[r] = jnp.dot(h, lw_ref[r], preferred_element_type=jnp.float32)
        gb_ref[r] = (jnp.dot(h, fw_ref[r], preferred_element_type=jnp.float32)
                     + fb_ref[r][None, :]) * inv_ref[:, r][:, None]
    bgs = jnp.dot(h, fsw_ref[...], preferred_element_type=jnp.float32)
    hs = jnp.dot(h, sw_ref[...], preferred_element_type=jnp.float32)
    so_ref[...] = jnp.maximum(bgs[:, HID:] * hs + bgs[:, :HID], 0.0)


def _tc_proj(h, lw, fw, fb, sw, fsw, inv4):
    grid = N_NODES // _NB
    return pl.pallas_call(
        _tc_proj_body,
        grid=(grid,),
        in_specs=[
            pl.BlockSpec((_NB, HID), lambda i: (i, 0)),
            pl.BlockSpec((N_REL, HID, HID), lambda i: (0, 0, 0)),
            pl.BlockSpec((N_REL, HID, 2 * HID), lambda i: (0, 0, 0)),
            pl.BlockSpec((N_REL, 2 * HID), lambda i: (0, 0)),
            pl.BlockSpec((HID, HID), lambda i: (0, 0)),
            pl.BlockSpec((HID, 2 * HID), lambda i: (0, 0)),
            pl.BlockSpec((_NB, N_REL), lambda i: (i, 0)),
        ],
        out_specs=[
            pl.BlockSpec((N_REL, _NB, HID), lambda i: (0, i, 0)),
            pl.BlockSpec((N_REL, _NB, 2 * HID), lambda i: (0, i, 0)),
            pl.BlockSpec((_NB, HID), lambda i: (i, 0)),
        ],
        out_shape=[
            _f32((N_REL, N_NODES, HID)),
            _f32((N_REL, N_NODES, 2 * HID)),
            _f32((N_NODES, HID)),
        ],
    )(h, lw, fw, fb, sw, fsw, inv4)


def _bn(hsum, w, b):
    m = jnp.mean(hsum, axis=0, keepdims=True)
    v = jnp.mean((hsum - m) * (hsum - m), axis=0, keepdims=True)
    return (hsum - m) * jax.lax.rsqrt(v + EPS) * w + b


def _tc_combine_body(so_ref, part_ref, w_ref, b_ref, out_ref):
    hsum = so_ref[...] + part_ref[0] + part_ref[1]
    out_ref[...] = _bn(hsum, w_ref[...], b_ref[...])


def _tc_combine(so, parts, w, b):
    return pl.pallas_call(
        _tc_combine_body,
        out_shape=_f32((N_NODES, HID)),
    )(so, parts, w, b)


def _tc_final_body(so_ref, part_ref, w_ref, b_ref, l1w_ref, l1b_ref,
                   l2w_ref, l2b_ref, out_ref):
    hsum = so_ref[...] + part_ref[0] + part_ref[1]
    h2 = _bn(hsum, w_ref[...], b_ref[...])
    t = jnp.dot(h2, l1w_ref[...], preferred_element_type=jnp.float32)
    t = t + l1b_ref[...]
    t = jnp.where(t > 0, t, 0.2 * t)
    out_ref[...] = (jnp.dot(t, l2w_ref[...],
                            preferred_element_type=jnp.float32)
                    + l2b_ref[...])


def _tc_final(so, parts, w, b, l1w, l1b, l2w, l2b):
    return pl.pallas_call(
        _tc_final_body,
        out_shape=_f32((N_NODES, l2w.shape[1])),
    )(so, parts, w, b, l1w, l1b, l2w, l2b)


def kernel(x, edge_index, edge_type, emb, lins_w, films_w, films_b,
           skip_w, film_skip_w, bn_w, bn_b, lin1_w, lin1_b, lin2_w, lin2_b):
    src = edge_index[0]
    dst = edge_index[1]
    h0, hridx, gbidx, inv_cnt = _sc_prep(src, dst, edge_type, x, emb)
    inv4 = inv_cnt[:N_REL * N_NODES].reshape(N_REL, N_NODES).T

    h = h0
    out = None
    n_layers = lins_w.shape[0]
    for l in range(n_layers):
        hr_all, gb_all, so = _tc_proj(h, lins_w[l], films_w[l], films_b[l],
                                      skip_w[l], film_skip_w[l], inv4)
        parts = _sc_edge(hr_all.reshape(N_REL * N_NODES, HID),
                         gb_all.reshape(N_REL * N_NODES, 2 * HID),
                         hridx, gbidx, dst)
        wl = bn_w[l].reshape(1, HID)
        bl = bn_b[l].reshape(1, HID)
        if l < n_layers - 1:
            h = _tc_combine(so, parts, wl, bl)
        else:
            out = _tc_final(so, parts, wl, bl, lin1_w,
                            lin1_b.reshape(1, -1), lin2_w,
                            lin2_b.reshape(1, -1))
    return out


# speedup vs baseline: 14.8327x; 1.5808x over previous
import functools

import jax
import jax.numpy as jnp
from jax import lax
from jax.experimental import pallas as pl
from jax.experimental.pallas import tpu as pltpu
from jax.experimental.pallas import tpu_sc as plsc

N_NODES = 10000
N_EDGES = 320000
HID = 128
N_REL = 4
EPS = 1e-5

NC = 2
NS = 16
NW = NC * NS
EPW = N_EDGES // NW
K = 40
NCHUNK = EPW // K
CNT_PAD = 40960
CSLICE = CNT_PAD // NS
RPT = N_NODES // NS
NPW = 312
E_CNT = N_EDGES // NS

_mesh = plsc.VectorSubcoreMesh(core_axis_name="c", subcore_axis_name="s")


def _spans16(n):
    s = [i * 16 for i in range(n // 16)]
    if n % 16:
        s.append(n - 16)
    return s


def _f32(shape):
    return jax.ShapeDtypeStruct(shape, jnp.float32)


def _i32(shape):
    return jax.ShapeDtypeStruct(shape, jnp.int32)


KP = 80
NCHUNK_P = EPW // KP


@functools.partial(
    pl.kernel,
    out_type=(
        _f32((N_NODES, HID)),
        _i32((N_EDGES,)),
        _i32((N_EDGES,)),
        _f32((NC, CNT_PAD)),
    ),
    mesh=_mesh,
    scratch_types=[
        pltpu.VMEM((NPW + 16,), jnp.int32),
        pltpu.VMEM((NPW + 16, HID), jnp.float32),
        pltpu.VMEM((KP,), jnp.int32),
        pltpu.VMEM((KP,), jnp.int32),
        pltpu.VMEM((KP,), jnp.int32),
        pltpu.VMEM((KP,), jnp.int32),
        pltpu.VMEM((KP,), jnp.int32),
        pltpu.VMEM((KP,), jnp.int32),
        pltpu.VMEM((KP,), jnp.int32),
        pltpu.VMEM((KP,), jnp.int32),
        pltpu.VMEM((KP,), jnp.int32),
        pltpu.VMEM((KP,), jnp.int32),
        pltpu.VMEM((KP,), jnp.int32),
        pltpu.VMEM((KP,), jnp.int32),
        pltpu.VMEM((KP,), jnp.float32),
        pltpu.VMEM_SHARED((CNT_PAD,), jnp.float32),
        pltpu.SemaphoreType.DMA,
        pltpu.SemaphoreType.DMA,
        pltpu.SemaphoreType.DMA,
        pltpu.SemaphoreType.DMA,
        pltpu.SemaphoreType.DMA,
        pltpu.SemaphoreType.DMA,
        pltpu.SemaphoreType.DMA,
    ],
)
def _sc_prep(src_hbm, dst_hbm, et_hbm, x_hbm, emb_hbm,
             h0_hbm, hridx_hbm, gbidx_hbm, cnt_hbm,
             xv, rows, srcv0, srcv1, dstv0, dstv1, typev0, typev1,
             hrbuf0, hrbuf1, gbbuf0, gbbuf1, gbscat0, gbscat1, onesv,
             cnt_sh, sem, sin0, sin1, sout0, sout1, ssc0, ssc1):
    c = lax.axis_index("c")
    s = lax.axis_index("s")
    wid = c * NS + s
    bufs = (
        (srcv0, dstv0, typev0, hrbuf0, gbbuf0, gbscat0, sin0, sout0, ssc0),
        (srcv1, dstv1, typev1, hrbuf1, gbbuf1, gbscat1, sin1, sout1, ssc1),
    )
    ebase = wid * EPW

    for j in range(HID // 16):
        rows[0, pl.ds(j * 16, 16)] = jnp.zeros((16,), jnp.float32)

    def zc(q, _):
        pltpu.sync_copy(rows.at[0, pl.ds(0, HID)],
                        cnt_sh.at[pl.ds(s * CSLICE + q * HID, HID)])
        return 0
    lax.fori_loop(0, CSLICE // HID, zc, 0)

    for g0 in _spans16(KP):
        onesv[pl.ds(g0, 16)] = jnp.ones((16,), jnp.float32)

    nbase = wid * NPW
    pltpu.sync_copy(x_hbm.at[pl.ds(nbase, NPW)], xv.at[pl.ds(0, NPW)])
    pltpu.async_copy(emb_hbm.at[xv.at[pl.ds(0, NPW)]],
                     rows.at[pl.ds(0, NPW)], sem).wait()
    pltpu.sync_copy(rows.at[pl.ds(0, NPW)], h0_hbm.at[pl.ds(nbase, NPW)])

    @pl.when(wid == 0)
    def _():
        tail = N_NODES - NW * NPW
        tb = NW * NPW
        pltpu.sync_copy(x_hbm.at[pl.ds(tb, tail)], xv.at[pl.ds(0, tail)])
        pltpu.async_copy(emb_hbm.at[xv.at[pl.ds(0, tail)]],
                         rows.at[pl.ds(0, tail)], sem).wait()
        pltpu.sync_copy(rows.at[pl.ds(0, tail)], h0_hbm.at[pl.ds(tb, tail)])

    plsc.subcore_barrier()

    def issue_in(i, b):
        srcv, dstv, typev = bufs[b][0], bufs[b][1], bufs[b][2]
        sin = bufs[b][6]
        base = ebase + i * KP
        pltpu.async_copy(src_hbm.at[pl.ds(base, KP)], srcv, sin)
        pltpu.async_copy(dst_hbm.at[pl.ds(base, KP)], dstv, sin)
        pltpu.async_copy(et_hbm.at[pl.ds(base, KP)], typev, sin)

    def wait_in(i, b):
        srcv, dstv, typev = bufs[b][0], bufs[b][1], bufs[b][2]
        sin = bufs[b][6]
        base = ebase + i * KP
        pltpu.make_async_copy(src_hbm.at[pl.ds(base, KP)], srcv, sin).wait()
        pltpu.make_async_copy(dst_hbm.at[pl.ds(base, KP)], dstv, sin).wait()
        pltpu.make_async_copy(et_hbm.at[pl.ds(base, KP)], typev, sin).wait()

    def drain_outs(i, b):
        hrbuf, gbbuf, gbscat = bufs[b][3], bufs[b][4], bufs[b][5]
        sout, ssc = bufs[b][7], bufs[b][8]
        base = ebase + i * KP
        pltpu.make_async_copy(hrbuf, hridx_hbm.at[pl.ds(base, KP)],
                              sout).wait()
        pltpu.make_async_copy(gbbuf, gbidx_hbm.at[pl.ds(base, KP)],
                              sout).wait()
        pltpu.make_async_copy(onesv, cnt_sh.at[gbscat], ssc).wait()

    def process(i, b, drain_pred, pre_in=True):
        srcv, dstv, typev, hrbuf, gbbuf, gbscat, sin, sout, ssc = bufs[b]
        wait_in(i, b)
        if drain_pred is True:
            drain_outs(i - 2, b)
        elif drain_pred is not False:
            pl.when(drain_pred)(lambda: drain_outs(i - 2, b))
        for g0 in _spans16(KP):
            sl = pl.ds(g0, 16)
            t = typev[sl]
            hrbuf[sl] = t * N_NODES + srcv[sl]
            g = t * N_NODES + dstv[sl]
            gbbuf[sl] = g
            gbscat[sl] = g
        if pre_in:
            issue_in(i + 2, b)
        base = ebase + i * KP
        pltpu.async_copy(hrbuf, hridx_hbm.at[pl.ds(base, KP)], sout)
        pltpu.async_copy(gbbuf, gbidx_hbm.at[pl.ds(base, KP)], sout)
        pltpu.async_copy(onesv, cnt_sh.at[gbscat], ssc, add=True)

    issue_in(0, 0)
    issue_in(1, 1)

    def pairp(ip, _):
        process(2 * ip, 0, ip > 0)
        process(2 * ip + 1, 1, ip > 0)
        return 0
    lax.fori_loop(0, (NCHUNK_P - 3) // 2, pairp, 0)

    process(NCHUNK_P - 3, 0, True, pre_in=True)
    process(NCHUNK_P - 2, 1, True, pre_in=False)
    process(NCHUNK_P - 1, 0, True, pre_in=False)
    drain_outs(NCHUNK_P - 2, 1)
    drain_outs(NCHUNK_P - 1, 0)

    plsc.subcore_barrier()
    pltpu.sync_copy(cnt_sh.at[pl.ds(s * CSLICE, CSLICE)],
                    cnt_hbm.at[c, pl.ds(s * CSLICE, CSLICE)])


@functools.partial(
    pl.kernel,
    out_type=_f32((NC, N_NODES, HID)),
    mesh=_mesh,
    scratch_types=[
        pltpu.VMEM((K, HID), jnp.float32),
        pltpu.VMEM((K, HID), jnp.float32),
        pltpu.VMEM((K, 2 * HID), jnp.float32),
        pltpu.VMEM((K, 2 * HID), jnp.float32),
        pltpu.VMEM((K, HID), jnp.float32),
        pltpu.VMEM((K, HID), jnp.float32),
        pltpu.VMEM((K,), jnp.int32),
        pltpu.VMEM((K,), jnp.int32),
        pltpu.VMEM((K,), jnp.int32),
        pltpu.VMEM((K,), jnp.int32),
        pltpu.VMEM((K,), jnp.int32),
        pltpu.VMEM((K,), jnp.int32),
        pltpu.VMEM((K,), jnp.int32),
        pltpu.VMEM((K,), jnp.int32),
        pltpu.VMEM((16, HID), jnp.float32),
        pltpu.VMEM_SHARED((N_NODES, HID), jnp.float32),
        pltpu.SemaphoreType.DMA,
        pltpu.SemaphoreType.DMA,
        pltpu.SemaphoreType.DMA,
        pltpu.SemaphoreType.DMA,
        pltpu.SemaphoreType.DMA,
        pltpu.SemaphoreType.DMA,
        pltpu.SemaphoreType.DMA,
        pltpu.SemaphoreType.DMA,
    ],
)
def _sc_edge(hr_hbm, gb_hbm, hridx_hbm, gbidx_hbm, dst_hbm,
             out_hbm,
             hrv0, hrv1, gbv0, gbv1, msgv0, msgv1,
             hrS0, hrS1, gbS0, gbS1, dstS0, dstS1, dstv0, dstv1,
             zrow, acc_sh,
             shr0, shr1, sgb0, sgb1, sidx0, sidx1, ssc0, ssc1):
    c = lax.axis_index("c")
    s = lax.axis_index("s")
    wid = c * NS + s
    bufs = (
        (hrv0, gbv0, msgv0, hrS0, gbS0, dstS0, dstv0, shr0, sgb0, sidx0,
         ssc0),
        (hrv1, gbv1, msgv1, hrS1, gbS1, dstS1, dstv1, shr1, sgb1, sidx1,
         ssc1),
    )

    rbase = s * 624
    nch = jnp.where(s == NS - 1, 40, 39)

    ebase = wid * EPW

    def zfill(r, _):
        for j in range(HID // 16):
            zrow[r, pl.ds(j * 16, 16)] = jnp.zeros((16,), jnp.float32)
        return 0
    lax.fori_loop(0, 16, zfill, 0)

    def zcopy(q, _):
        pltpu.sync_copy(zrow, acc_sh.at[pl.ds(rbase + q * 16, 16)])
        return 0
    lax.fori_loop(0, nch, zcopy, 0)
    plsc.subcore_barrier()

    def issue_idx(i, b):
        _, _, _, hrS, gbS, dstS, _, _, _, sidx, _ = bufs[b]
        sl = pl.ds(ebase + i * K, K)
        pltpu.async_copy(hridx_hbm.at[sl], hrS, sidx)
        pltpu.async_copy(gbidx_hbm.at[sl], gbS, sidx)
        pltpu.async_copy(dst_hbm.at[sl], dstS, sidx)

    def wait_idx(i, b):
        _, _, _, hrS, gbS, dstS, _, _, _, sidx, _ = bufs[b]
        sl = pl.ds(ebase + i * K, K)
        pltpu.make_async_copy(hridx_hbm.at[sl], hrS, sidx).wait()
        pltpu.make_async_copy(gbidx_hbm.at[sl], gbS, sidx).wait()
        pltpu.make_async_copy(dst_hbm.at[sl], dstS, sidx).wait()

    def issue_gathers(b):
        hrv, gbv, _, hrS, gbS, _, _, shr, sgb, _, _ = bufs[b]
        pltpu.async_copy(hr_hbm.at[hrS], hrv, shr)
        pltpu.async_copy(gb_hbm.at[gbS], gbv, sgb)

    def wait_gathers(b):
        hrv, gbv, _, hrS, gbS, _, _, shr, sgb, _, _ = bufs[b]
        pltpu.make_async_copy(hr_hbm.at[hrS], hrv, shr).wait()
        pltpu.make_async_copy(gb_hbm.at[gbS], gbv, sgb).wait()

    def wait_scat(b):
        msgv, dstv, ssc = bufs[b][2], bufs[b][6], bufs[b][10]
        pltpu.make_async_copy(msgv, acc_sh.at[dstv], ssc).wait()

    def process(i, b, scat_pred, pre_idx=True, pre_gather=True):
        hrv, gbv, msgv, hrS, gbS, dstS, dstv, shr, sgb, sidx, ssc = bufs[b]
        if pre_gather:
            wait_idx(i + 1, 1 - b)
            issue_gathers(1 - b)
        wait_gathers(b)
        if scat_pred is True:
            wait_scat(b)
        elif scat_pred is not False:
            pl.when(scat_pred)(lambda: wait_scat(b))
        for g0 in _spans16(K):
            gsl = pl.ds(g0, 16)
            dstv[gsl] = dstS[gsl]
        if pre_idx:
            issue_idx(i + 2, b)

        def ebody(e, _):
            for j in range(HID // 16):
                fsl = pl.ds(j * 16, 16)
                beta = gbv[e, fsl]
                gamma = gbv[e, pl.ds(HID + j * 16, 16)]
                hr = hrv[e, fsl]
                msgv[e, fsl] = jnp.maximum(gamma * hr + beta, 0.0)
            return 0
        lax.fori_loop(0, K, ebody, 0)

        pltpu.async_copy(msgv, acc_sh.at[dstv], ssc, add=True)

    issue_idx(0, 0)
    wait_idx(0, 0)
    issue_gathers(0)
    issue_idx(1, 1)

    def pair(ip, _):
        process(2 * ip, 0, ip > 0)
        process(2 * ip + 1, 1, ip > 0)
        return 0
    lax.fori_loop(0, NCHUNK // 2 - 1, pair, 0)

    process(NCHUNK - 2, 0, True, pre_idx=False, pre_gather=True)
    process(NCHUNK - 1, 1, True, pre_idx=False, pre_gather=False)
    wait_scat(0)
    wait_scat(1)

    plsc.subcore_barrier()

    def ocopy(q, _):
        rb = rbase + q * 16
        pltpu.sync_copy(acc_sh.at[pl.ds(rb, 16)], zrow)
        pltpu.sync_copy(zrow, out_hbm.at[c, pl.ds(rb, 16)])
        return 0
    lax.fori_loop(0, nch, ocopy, 0)


_NB = 1000


def _tc_proj_body(h_ref, lw_ref, fw_ref, fb_ref, sw_ref, fsw_ref, cnt_ref,
                  hr_ref, gb_ref, so_ref):
    h = h_ref[...]
    cc = cnt_ref[...]
    inv = 1.0 / jnp.maximum(cc[:, :, 0] + cc[:, :, 1], 1.0)
    for r in range(N_REL):
        hr_ref[r] = jnp.dot(h, lw_ref[r], preferred_element_type=jnp.float32)
        gb_ref[r] = (jnp.dot(h, fw_ref[r], preferred_element_type=jnp.float32)
                     + fb_ref[r][None, :]) * inv[:, r][:, None]
    bgs = jnp.dot(h, fsw_ref[...], preferred_element_type=jnp.float32)
    hs = jnp.dot(h, sw_ref[...], preferred_element_type=jnp.float32)
    so_ref[...] = jnp.maximum(bgs[:, HID:] * hs + bgs[:, :HID], 0.0)


def _tc_proj(h, lw, fw, fb, sw, fsw, cnt3):
    grid = N_NODES // _NB
    return pl.pallas_call(
        _tc_proj_body,
        grid=(grid,),
        in_specs=[
            pl.BlockSpec((_NB, HID), lambda i: (i, 0)),
            pl.BlockSpec((N_REL, HID, HID), lambda i: (0, 0, 0)),
            pl.BlockSpec((N_REL, HID, 2 * HID), lambda i: (0, 0, 0)),
            pl.BlockSpec((N_REL, 2 * HID), lambda i: (0, 0)),
            pl.BlockSpec((HID, HID), lambda i: (0, 0)),
            pl.BlockSpec((HID, 2 * HID), lambda i: (0, 0)),
            pl.BlockSpec((_NB, N_REL, NC), lambda i: (i, 0, 0)),
        ],
        out_specs=[
            pl.BlockSpec((N_REL, _NB, HID), lambda i: (0, i, 0)),
            pl.BlockSpec((N_REL, _NB, 2 * HID), lambda i: (0, i, 0)),
            pl.BlockSpec((_NB, HID), lambda i: (i, 0)),
        ],
        out_shape=[
            _f32((N_REL, N_NODES, HID)),
            _f32((N_REL, N_NODES, 2 * HID)),
            _f32((N_NODES, HID)),
        ],
    )(h, lw, fw, fb, sw, fsw, cnt3)


def _bn(hsum, w, b):
    m = jnp.mean(hsum, axis=0, keepdims=True)
    v = jnp.mean((hsum - m) * (hsum - m), axis=0, keepdims=True)
    return (hsum - m) * jax.lax.rsqrt(v + EPS) * w + b


def _tc_combine_body(so_ref, part_ref, w_ref, b_ref, out_ref):
    hsum = so_ref[...] + part_ref[0] + part_ref[1]
    out_ref[...] = _bn(hsum, w_ref[...], b_ref[...])


def _tc_combine(so, parts, w, b):
    return pl.pallas_call(
        _tc_combine_body,
        out_shape=_f32((N_NODES, HID)),
    )(so, parts, w, b)


def _tc_final_body(so_ref, part_ref, w_ref, b_ref, l1w_ref, l1b_ref,
                   l2w_ref, l2b_ref, out_ref):
    hsum = so_ref[...] + part_ref[0] + part_ref[1]
    h2 = _bn(hsum, w_ref[...], b_ref[...])
    t = jnp.dot(h2, l1w_ref[...], preferred_element_type=jnp.float32)
    t = t + l1b_ref[...]
    t = jnp.where(t > 0, t, 0.2 * t)
    out_ref[...] = (jnp.dot(t, l2w_ref[...],
                            preferred_element_type=jnp.float32)
                    + l2b_ref[...])


def _tc_final(so, parts, w, b, l1w, l1b, l2w, l2b):
    return pl.pallas_call(
        _tc_final_body,
        out_shape=_f32((N_NODES, l2w.shape[1])),
    )(so, parts, w, b, l1w, l1b, l2w, l2b)


def kernel(x, edge_index, edge_type, emb, lins_w, films_w, films_b,
           skip_w, film_skip_w, bn_w, bn_b, lin1_w, lin1_b, lin2_w, lin2_b):
    src = edge_index[0]
    dst = edge_index[1]
    h0, hridx, gbidx, cnt_parts = _sc_prep(src, dst, edge_type, x, emb)
    cnt3 = (cnt_parts[:, :N_REL * N_NODES]
            .reshape(NC, N_REL, N_NODES).transpose(2, 1, 0))

    h = h0
    out = None
    n_layers = lins_w.shape[0]
    for l in range(n_layers):
        hr_all, gb_all, so = _tc_proj(h, lins_w[l], films_w[l], films_b[l],
                                      skip_w[l], film_skip_w[l], cnt3)
        parts = _sc_edge(hr_all.reshape(N_REL * N_NODES, HID),
                         gb_all.reshape(N_REL * N_NODES, 2 * HID),
                         hridx, gbidx, dst)
        wl = bn_w[l].reshape(1, HID)
        bl = bn_b[l].reshape(1, HID)
        if l < n_layers - 1:
            h = _tc_combine(so, parts, wl, bl)
        else:
            out = _tc_final(so, parts, wl, bl, lin1_w,
                            lin1_b.reshape(1, -1), lin2_w,
                            lin2_b.reshape(1, -1))
    return out
